# baseline (speedup 1.0000x reference)
import jax
import jax.numpy as jnp
from jax import lax
from jax.experimental import pallas as pl
from jax.experimental.pallas import tpu as pltpu
from jax.experimental.pallas import tpu_sc as plsc

L = 16
NC = 2
NS = 16
NW = NC * NS
N_COMP = 5
C_DIM = 64
NSLOT = 3


def _make_body(n_comp, per_w):
    def body(x_hbm, comp_hbm, *rest):
        embs = rest[:n_comp]
        outs = rest[n_comp:2 * n_comp]
        x_v, cidx_v, cvals_v, rows_v, semc, semg, semw = rest[2 * n_comp:]
        t_vocab = comp_hbm.shape[0] // n_comp
        nslot = min(NSLOT, n_comp)

        wid = lax.axis_index("s") * NC + lax.axis_index("c")
        base = wid * per_w

        pltpu.sync_copy(x_hbm.at[pl.ds(base, per_w)], x_v)
        for i in range(n_comp):
            off = jnp.int32(i * t_vocab)
            for j in range(per_w // L):
                cidx_v[pl.ds(i * per_w + j * L, L)] = x_v[pl.ds(j * L, L)] + off

        cg = [
            pltpu.async_copy(
                comp_hbm.at[cidx_v.at[pl.ds(i * per_w, per_w)]],
                cvals_v.at[pl.ds(i * per_w, per_w)], semc.at[i])
            for i in range(n_comp)
        ]

        gd = [None] * n_comp
        wd = [None] * n_comp

        def fire(i):
            gd[i] = pltpu.async_copy(
                embs[i].at[cvals_v.at[pl.ds(i * per_w, per_w)]],
                rows_v.at[i % nslot], semg.at[i % nslot])

        def fire_wb(i):
            wd[i] = pltpu.async_copy(
                rows_v.at[i % nslot], outs[i].at[pl.ds(base, per_w)],
                semw.at[i % nslot])

        for i in range(n_comp):
            if i >= nslot:
                wd[i - nslot].wait()
            cg[i].wait()
            fire(i)
            if i >= 1:
                gd[i - 1].wait()
                fire_wb(i - 1)
        gd[n_comp - 1].wait()
        fire_wb(n_comp - 1)
        for i in range(max(0, n_comp - nslot), n_comp):
            wd[i].wait()

    return body


def _sc_call(x, comp_cm, embs, batch, per_w):
    n_comp = len(embs)
    mesh = plsc.VectorSubcoreMesh(core_axis_name="c", subcore_axis_name="s")
    out_type = tuple(
        jax.ShapeDtypeStruct((batch, C_DIM), jnp.float32) for _ in range(n_comp)
    )
    nslot = min(NSLOT, n_comp)
    scratch = [
        pltpu.VMEM((per_w,), jnp.int32),
        pltpu.VMEM((n_comp * per_w,), jnp.int32),
        pltpu.VMEM((n_comp * per_w,), jnp.int32),
        pltpu.VMEM((nslot, per_w, C_DIM), jnp.float32),
        pltpu.SemaphoreType.DMA((n_comp,)),
        pltpu.SemaphoreType.DMA((nslot,)),
        pltpu.SemaphoreType.DMA((nslot,)),
    ]
    f = pl.kernel(
        _make_body(n_comp, per_w), mesh=mesh, out_type=out_type,
        scratch_types=scratch,
        compiler_params=pltpu.CompilerParams(use_tc_tiling_on_sc=False),
    )
    return f(x, comp_cm, *embs)


def kernel(x, comp_table, emb0, emb1, emb2, emb3, emb4):
    batch = x.shape[0]
    per_w = batch // NW
    t_vocab = comp_table.shape[0]
    srcs = (emb0, emb1, emb2, emb3, emb4)
    reps = [max(1, min(1024, 4096 // e.shape[0])) for e in srcs]
    embs = tuple(
        jnp.tile(e.at[0].set(0.0), (r, 1)) for e, r in zip(srcs, reps)
    )
    t_ids = jnp.arange(t_vocab, dtype=jnp.int32)
    cols = [
        comp_table[:, i] + (t_ids % reps[i]) * e.shape[0]
        for i, e in enumerate(srcs)
    ]
    groups = [(0, 2), (2, 4), (4, 5)]
    outs = ()
    for lo, hi in groups:
        cm = cols[lo] if hi - lo == 1 else jnp.concatenate(cols[lo:hi])
        outs += _sc_call(x, cm.reshape(-1), embs[lo:hi], batch, per_w)
    return outs


# speedup vs baseline: 1.0167x; 1.0167x over previous
import jax
import jax.numpy as jnp
from jax import lax
from jax.experimental import pallas as pl
from jax.experimental.pallas import tpu as pltpu
from jax.experimental.pallas import tpu_sc as plsc

L = 16
NC = 2
NS = 16
NW = NC * NS
N_COMP = 5
C_DIM = 64
NSLOT = 3


def _make_body(n_comp, per_w):
    def body(x_hbm, comp_hbm, *rest):
        embs = rest[:n_comp]
        outs = rest[n_comp:2 * n_comp]
        x_v, cidx_v, cvals_v, rows_v, semc, semg, semw = rest[2 * n_comp:]
        t_vocab = comp_hbm.shape[0] // n_comp
        nslot = min(NSLOT, n_comp)

        wid = lax.axis_index("s") * NC + lax.axis_index("c")
        base = wid * per_w

        pltpu.sync_copy(x_hbm.at[pl.ds(base, per_w)], x_v)
        for i in range(n_comp):
            off = jnp.int32(i * t_vocab)
            for j in range(per_w // L):
                cidx_v[pl.ds(i * per_w + j * L, L)] = x_v[pl.ds(j * L, L)] + off

        cg = [
            pltpu.async_copy(
                comp_hbm.at[cidx_v.at[pl.ds(i * per_w, per_w)]],
                cvals_v.at[pl.ds(i * per_w, per_w)], semc.at[i])
            for i in range(n_comp)
        ]

        gd = [None] * n_comp
        wd = [None] * n_comp

        def fire(i):
            gd[i] = pltpu.async_copy(
                embs[i].at[cvals_v.at[pl.ds(i * per_w, per_w)]],
                rows_v.at[i % nslot], semg.at[i % nslot])

        def fire_wb(i):
            wd[i] = pltpu.async_copy(
                rows_v.at[i % nslot], outs[i].at[pl.ds(base, per_w)],
                semw.at[i % nslot])

        for i in range(n_comp):
            if i >= nslot:
                wd[i - nslot].wait()
            cg[i].wait()
            fire(i)
            if i >= 1:
                gd[i - 1].wait()
                fire_wb(i - 1)
        gd[n_comp - 1].wait()
        fire_wb(n_comp - 1)
        for i in range(max(0, n_comp - nslot), n_comp):
            wd[i].wait()

    return body


def _sc_call(x, comp_cm, embs, batch, per_w):
    n_comp = len(embs)
    mesh = plsc.VectorSubcoreMesh(core_axis_name="c", subcore_axis_name="s")
    out_type = tuple(
        jax.ShapeDtypeStruct((batch, C_DIM), jnp.float32) for _ in range(n_comp)
    )
    nslot = min(NSLOT, n_comp)
    scratch = [
        pltpu.VMEM((per_w,), jnp.int32),
        pltpu.VMEM((n_comp * per_w,), jnp.int32),
        pltpu.VMEM((n_comp * per_w,), jnp.int32),
        pltpu.VMEM((nslot, per_w, C_DIM), jnp.float32),
        pltpu.SemaphoreType.DMA((n_comp,)),
        pltpu.SemaphoreType.DMA((nslot,)),
        pltpu.SemaphoreType.DMA((nslot,)),
    ]
    f = pl.kernel(
        _make_body(n_comp, per_w), mesh=mesh, out_type=out_type,
        scratch_types=scratch,
        compiler_params=pltpu.CompilerParams(use_tc_tiling_on_sc=False),
    )
    return f(x, comp_cm, *embs)


def kernel(x, comp_table, emb0, emb1, emb2, emb3, emb4):
    batch = x.shape[0]
    per_w = batch // NW
    t_vocab = comp_table.shape[0]
    srcs = (emb0, emb1, emb2, emb3, emb4)
    reps = [max(1, min(1024, 4096 // e.shape[0])) for e in srcs]
    embs = tuple(
        jnp.tile(e.at[0].set(0.0), (r, 1)) for e, r in zip(srcs, reps)
    )
    t_ids = jnp.arange(t_vocab, dtype=jnp.int32)
    cols = [
        comp_table[:, i] + (t_ids % reps[i]) * e.shape[0]
        for i, e in enumerate(srcs)
    ]
    groups = [(0, 3), (3, 5)]
    outs = ()
    for lo, hi in groups:
        cm = cols[lo] if hi - lo == 1 else jnp.concatenate(cols[lo:hi])
        outs += _sc_call(x, cm.reshape(-1), embs[lo:hi], batch, per_w)
    return outs
